# Initial kernel scaffold; baseline (speedup 1.0000x reference)
import jax
import jax.numpy as jnp
from jax.experimental import pallas as pl


def kernel(x, node_ids, time_ids, temporal_pe, spatial_pe):
    raise NotImplementedError("write your pallas kernel here")


# speedup vs baseline: 2.1547x; 2.1547x over previous
import jax
import jax.numpy as jnp
from jax import lax
from jax.experimental import pallas as pl
from jax.experimental.pallas import tpu as pltpu
from jax.experimental.pallas import tpu_sc as plsc

N = 100000
HALF = 128
OUT_D = 256
NC = 2
NS = 16
NW = NC * NS
C = 80
NCHUNK = N // C


def _pe_kernel(node_hbm, time_hbm, tpe_hbm, spe_hbm, out_hbm,
               nidx_v, tidx_v, trows_v, srows_v, tsem, ssem):
    wid = lax.axis_index("s") * NC + lax.axis_index("c")
    nloop = (NCHUNK - wid + NW - 1) // NW

    def body(j, carry):
        base = (wid + j * NW) * C
        pltpu.sync_copy(time_hbm.at[pl.ds(base, C)], tidx_v)
        pltpu.sync_copy(node_hbm.at[pl.ds(base, C)], nidx_v)
        ct = pltpu.async_copy(tpe_hbm.at[tidx_v], trows_v, tsem)
        cs = pltpu.async_copy(spe_hbm.at[nidx_v], srows_v, ssem)
        ct.wait()
        cs.wait()
        pltpu.sync_copy(trows_v, out_hbm.at[pl.ds(base, C), pl.ds(0, HALF)])
        pltpu.sync_copy(srows_v, out_hbm.at[pl.ds(base, C), pl.ds(HALF, HALF)])
        return carry

    lax.fori_loop(0, nloop, body, 0)


def kernel(x, node_ids, time_ids, temporal_pe, spatial_pe):
    del x
    mesh = plsc.VectorSubcoreMesh(core_axis_name="c", subcore_axis_name="s")
    f = pl.kernel(
        _pe_kernel,
        out_type=jax.ShapeDtypeStruct((N, OUT_D), jnp.float32),
        mesh=mesh,
        scratch_types=[
            pltpu.VMEM((C,), jnp.int32),
            pltpu.VMEM((C,), jnp.int32),
            pltpu.VMEM((C, HALF), jnp.float32),
            pltpu.VMEM((C, HALF), jnp.float32),
            pltpu.SemaphoreType.DMA,
            pltpu.SemaphoreType.DMA,
        ],
    )
    return f(node_ids, time_ids, temporal_pe, spatial_pe)


# speedup vs baseline: 2.8348x; 1.3156x over previous
import jax
import jax.numpy as jnp
from jax import lax
from jax.experimental import pallas as pl
from jax.experimental.pallas import tpu as pltpu
from jax.experimental.pallas import tpu_sc as plsc

N = 100000
HALF = 128
OUT_D = 256
NC = 2
NS = 16
NW = NC * NS
C = 400
NCHUNK = N // C


def _pe_kernel(node_hbm, time_hbm, tpe_hbm, spe_hbm, out_hbm,
               nidx_v, tidx_v, trows_v, srows_v, tsem, ssem):
    wid = lax.axis_index("s") * NC + lax.axis_index("c")
    nloop = (NCHUNK - wid + NW - 1) // NW

    def body(j, carry):
        base = (wid + j * NW) * C
        pltpu.sync_copy(time_hbm.at[pl.ds(base, C)], tidx_v)
        pltpu.sync_copy(node_hbm.at[pl.ds(base, C)], nidx_v)
        ct = pltpu.async_copy(tpe_hbm.at[tidx_v], trows_v, tsem)
        cs = pltpu.async_copy(spe_hbm.at[nidx_v], srows_v, ssem)
        ct.wait()
        cs.wait()
        pltpu.sync_copy(trows_v, out_hbm.at[pl.ds(base, C), pl.ds(0, HALF)])
        pltpu.sync_copy(srows_v, out_hbm.at[pl.ds(base, C), pl.ds(HALF, HALF)])
        return carry

    lax.fori_loop(0, nloop, body, 0)


def kernel(x, node_ids, time_ids, temporal_pe, spatial_pe):
    del x
    mesh = plsc.VectorSubcoreMesh(core_axis_name="c", subcore_axis_name="s")
    f = pl.kernel(
        _pe_kernel,
        out_type=jax.ShapeDtypeStruct((N, OUT_D), jnp.float32),
        mesh=mesh,
        scratch_types=[
            pltpu.VMEM((C,), jnp.int32),
            pltpu.VMEM((C,), jnp.int32),
            pltpu.VMEM((C, HALF), jnp.float32),
            pltpu.VMEM((C, HALF), jnp.float32),
            pltpu.SemaphoreType.DMA,
            pltpu.SemaphoreType.DMA,
        ],
    )
    return f(node_ids, time_ids, temporal_pe, spatial_pe)


# speedup vs baseline: 2.8627x; 1.0098x over previous
import jax
import jax.numpy as jnp
from jax import lax
from jax.experimental import pallas as pl
from jax.experimental.pallas import tpu as pltpu
from jax.experimental.pallas import tpu_sc as plsc

N = 100000
HALF = 128
OUT_D = 256
NC = 2
NS = 16
NW = NC * NS
C = 200
NCHUNK = N // C
J = -(-NCHUNK // NW)
LAST_FULL = NCHUNK - (J - 1) * NW


def _pe_kernel(node_hbm, time_hbm, tpe_hbm, spe_hbm, out_hbm, *scratch):
    nidx = scratch[0:2]
    tidx = scratch[2:4]
    trows = scratch[4:6]
    srows = scratch[6:8]
    gt, gs, wt, ws = scratch[8:10], scratch[10:12], scratch[12:14], scratch[14:16]

    wid = lax.axis_index("s") * NC + lax.axis_index("c")
    has_last = wid < LAST_FULL

    def descs(j):
        b = j % 2
        base = (wid + j * NW) * C
        return (
            pltpu.make_async_copy(tpe_hbm.at[tidx[b]], trows[b], gt[b]),
            pltpu.make_async_copy(spe_hbm.at[nidx[b]], srows[b], gs[b]),
            pltpu.make_async_copy(
                trows[b], out_hbm.at[pl.ds(base, C), pl.ds(0, HALF)], wt[b]),
            pltpu.make_async_copy(
                srows[b], out_hbm.at[pl.ds(base, C), pl.ds(HALF, HALF)], ws[b]),
        )

    d = [descs(j) for j in range(J)]

    def issue_gathers(j):
        b = j % 2
        base = (wid + j * NW) * C
        pltpu.sync_copy(time_hbm.at[pl.ds(base, C)], tidx[b])
        pltpu.sync_copy(node_hbm.at[pl.ds(base, C)], nidx[b])
        d[j][0].start()
        d[j][1].start()

    issue_gathers(0)
    for j in range(J):
        if j + 1 < J:
            def lookahead(jj=j):
                if jj >= 1:
                    d[jj - 1][2].wait()
                    d[jj - 1][3].wait()
                issue_gathers(jj + 1)
            if j + 1 == J - 1:
                pl.when(has_last)(lookahead)
            else:
                lookahead()

        def finish(jj=j):
            d[jj][0].wait()
            d[jj][1].wait()
            d[jj][2].start()
            d[jj][3].start()
        if j == J - 1:
            pl.when(has_last)(finish)
        else:
            finish()

    def drain_last():
        d[J - 1][2].wait()
        d[J - 1][3].wait()
    def drain_prev():
        d[J - 3][2].wait()
        d[J - 3][3].wait()
    pl.when(has_last)(drain_last)
    pl.when(jnp.logical_not(has_last))(drain_prev)
    d[J - 2][2].wait()
    d[J - 2][3].wait()


def kernel(x, node_ids, time_ids, temporal_pe, spatial_pe):
    del x
    mesh = plsc.VectorSubcoreMesh(core_axis_name="c", subcore_axis_name="s")
    f = pl.kernel(
        _pe_kernel,
        out_type=jax.ShapeDtypeStruct((N, OUT_D), jnp.float32),
        mesh=mesh,
        scratch_types=(
            [pltpu.VMEM((C,), jnp.int32) for _ in range(4)]
            + [pltpu.VMEM((C, HALF), jnp.float32) for _ in range(4)]
            + [pltpu.SemaphoreType.DMA for _ in range(8)]
        ),
    )
    return f(node_ids, time_ids, temporal_pe, spatial_pe)
